# baseline (speedup 1.0000x reference)
import functools
import math

import jax
import jax.numpy as jnp
from jax import lax
from jax.experimental import pallas as pl
from jax.experimental.pallas import tpu as pltpu
from jax.experimental.pallas import tpu_sc as plsc

D_DIM = 1024
SCALE = 1.0 / math.sqrt(D_DIM)
NCORES = 2
NSUB = 16
NWORK = NCORES * NSUB
CHUNK_T = 8
NBUF = 3
AHEAD = 2
S_SC = 512
TBLK = 512


def _sc_rows(x, table, idx_sc, S, B, D, t_lo):
    spw = S // NWORK
    nchunk = spw // CHUNK_T
    n_main = (nchunk // NBUF) * NBUF
    n_outer = n_main // NBUF
    groups = D // 16

    mesh = plsc.VectorSubcoreMesh(core_axis_name="c", subcore_axis_name="s")

    @functools.partial(
        pl.kernel,
        out_type=jax.ShapeDtypeStruct((t_lo + S, B, D), jnp.float32),
        mesh=mesh,
        scratch_types=(
            [pltpu.VMEM((spw,), jnp.int32),
             pltpu.VMEM((NBUF, CHUNK_T, B, D), jnp.float32),
             pltpu.VMEM((NBUF, CHUNK_T, D), jnp.float32)]
            + [pltpu.SemaphoreType.DMA] * (3 * NBUF)
        ),
    )
    def run(x_hbm, t_hbm, idx_hbm, o_hbm, idx_v, bufs, pes, *sems):
        sem_x = sems[0:NBUF]
        sem_pe = sems[NBUF:2 * NBUF]
        sem_o = sems[2 * NBUF:3 * NBUF]
        wid = lax.axis_index("s") * NCORES + lax.axis_index("c")
        t_base = wid * spw

        def in_start(c, s):
            tstart = t_base + c * CHUNK_T
            pltpu.async_copy(
                x_hbm.at[pl.ds(t_lo + tstart, CHUNK_T)], bufs.at[s],
                sem_x[s])
            pltpu.async_copy(
                t_hbm.at[idx_v.at[pl.ds(c * CHUNK_T, CHUNK_T)]],
                pes.at[s], sem_pe[s])

        def in_wait(s):
            pltpu.make_async_copy(
                x_hbm.at[pl.ds(0, CHUNK_T)], bufs.at[s], sem_x[s]).wait()
            pltpu.make_async_copy(
                t_hbm.at[pl.ds(0, CHUNK_T)], pes.at[s], sem_pe[s]).wait()

        def out_start(c, s):
            tstart = t_base + c * CHUNK_T
            pltpu.async_copy(
                bufs.at[s], o_hbm.at[pl.ds(t_lo + tstart, CHUNK_T)],
                sem_o[s])

        def out_wait(s):
            pltpu.make_async_copy(
                bufs.at[s], o_hbm.at[pl.ds(0, CHUNK_T)], sem_o[s]).wait()

        def compute(s):
            @pl.loop(0, CHUNK_T * groups, unroll=4)
            def body(j):
                t = j // groups
                g = j - t * groups
                sl = pl.ds(g * 16, 16)
                v = pes[s, t, sl] * SCALE
                for b in range(B):
                    plsc.addupdate(bufs.at[s, t, b, sl], v)

        def step(c, s, guard):
            s2 = (s + AHEAD) % NBUF
            if guard:
                @pl.when(c + AHEAD < nchunk)
                def _issue():
                    @pl.when(c + AHEAD >= NBUF)
                    def _drain():
                        out_wait(s2)
                    in_start(c + AHEAD, s2)
            else:
                if c + AHEAD < nchunk:
                    if c + AHEAD >= NBUF:
                        out_wait(s2)
                    in_start(c + AHEAD, s2)
            in_wait(s)
            compute(s)
            out_start(c, s)

        pltpu.sync_copy(idx_hbm.at[pl.ds(t_base, spw)], idx_v)
        for s in range(AHEAD):
            in_start(s, s)

        @pl.loop(0, n_outer)
        def outer(k):
            for i in range(NBUF):
                step(k * NBUF + i, i, True)

        for c in range(n_main, nchunk):
            step(c, c % NBUF, False)

        for c in range(max(0, nchunk - NBUF), nchunk):
            out_wait(c % NBUF)

    return run(x, table, idx_sc)


def _tc_body(pad_ref, x_ref, t_ref, prev_ref, o_ref):
    del pad_ref, prev_ref
    o_ref[...] = x_ref[...] + t_ref[...][:, None, :] * SCALE


def kernel(x, table, padding):
    T, B, D = x.shape
    n_rows = table.shape[0]
    t_lo = T - S_SC
    pad32 = jnp.asarray(padding, jnp.int32)
    idx_sc = jnp.clip(
        jnp.arange(t_lo, T, dtype=jnp.int32) + pad32, 0, n_rows - 1)

    sc_out = _sc_rows(x, table, idx_sc, S_SC, B, D, t_lo)

    tb = TBLK
    pad = pad32.reshape((1,))

    def x_map(i, pad_ref):
        del pad_ref
        return (i, 0, 0)

    def t_map(i, pad_ref):
        blk = jnp.minimum(i + pad_ref[0] // tb, n_rows // tb - 1)
        return (blk, 0)

    tc_out = pl.pallas_call(
        _tc_body,
        grid_spec=pltpu.PrefetchScalarGridSpec(
            num_scalar_prefetch=1,
            grid=(t_lo // tb,),
            in_specs=[
                pl.BlockSpec((tb, B, D), x_map),
                pl.BlockSpec((tb, D), t_map),
                pl.BlockSpec(memory_space=pl.ANY),
            ],
            out_specs=pl.BlockSpec((tb, B, D), x_map),
        ),
        out_shape=jax.ShapeDtypeStruct(x.shape, x.dtype),
        input_output_aliases={3: 0},
        compiler_params=pltpu.CompilerParams(
            dimension_semantics=("arbitrary",),
        ),
    )(pad, x, table, sc_out)

    return tc_out


# speedup vs baseline: 1.0178x; 1.0178x over previous
import functools
import math

import jax
import jax.numpy as jnp
from jax import lax
from jax.experimental import pallas as pl
from jax.experimental.pallas import tpu as pltpu
from jax.experimental.pallas import tpu_sc as plsc

D_DIM = 1024
SCALE = 1.0 / math.sqrt(D_DIM)
NCORES = 2
NSUB = 16
NWORK = NCORES * NSUB
CHUNK_T = 8
NBUF = 3
AHEAD = 2
S_SC = 512
TBLK = 512


def _sc_rows(x, table, idx_sc, S, B, D, t_lo):
    spw = S // NWORK
    nchunk = spw // CHUNK_T
    n_main = (nchunk // NBUF) * NBUF
    n_outer = n_main // NBUF
    groups = D // 16

    mesh = plsc.VectorSubcoreMesh(core_axis_name="c", subcore_axis_name="s")

    @functools.partial(
        pl.kernel,
        out_type=jax.ShapeDtypeStruct((S, B, D), jnp.float32),
        mesh=mesh,
        scratch_types=(
            [pltpu.VMEM((spw,), jnp.int32),
             pltpu.VMEM((NBUF, CHUNK_T, B, D), jnp.float32),
             pltpu.VMEM((NBUF, CHUNK_T, D), jnp.float32)]
            + [pltpu.SemaphoreType.DMA] * (3 * NBUF)
        ),
    )
    def run(x_hbm, t_hbm, idx_hbm, o_hbm, idx_v, bufs, pes, *sems):
        sem_x = sems[0:NBUF]
        sem_pe = sems[NBUF:2 * NBUF]
        sem_o = sems[2 * NBUF:3 * NBUF]
        wid = lax.axis_index("s") * NCORES + lax.axis_index("c")
        t_base = wid * spw

        def in_start(c, s):
            tstart = t_base + c * CHUNK_T
            pltpu.async_copy(
                x_hbm.at[pl.ds(t_lo + tstart, CHUNK_T)], bufs.at[s],
                sem_x[s])
            pltpu.async_copy(
                t_hbm.at[idx_v.at[pl.ds(c * CHUNK_T, CHUNK_T)]],
                pes.at[s], sem_pe[s])

        def in_wait(s):
            pltpu.make_async_copy(
                x_hbm.at[pl.ds(0, CHUNK_T)], bufs.at[s], sem_x[s]).wait()
            pltpu.make_async_copy(
                t_hbm.at[pl.ds(0, CHUNK_T)], pes.at[s], sem_pe[s]).wait()

        def out_start(c, s):
            tstart = t_base + c * CHUNK_T
            pltpu.async_copy(
                bufs.at[s], o_hbm.at[pl.ds(tstart, CHUNK_T)], sem_o[s])

        def out_wait(s):
            pltpu.make_async_copy(
                bufs.at[s], o_hbm.at[pl.ds(0, CHUNK_T)], sem_o[s]).wait()

        def compute(s):
            @pl.loop(0, CHUNK_T * groups, unroll=4)
            def body(j):
                t = j // groups
                g = j - t * groups
                sl = pl.ds(g * 16, 16)
                v = pes[s, t, sl] * SCALE
                for b in range(B):
                    plsc.addupdate(bufs.at[s, t, b, sl], v)

        def step(c, s, guard):
            s2 = (s + AHEAD) % NBUF
            if guard:
                @pl.when(c + AHEAD < nchunk)
                def _issue():
                    @pl.when(c + AHEAD >= NBUF)
                    def _drain():
                        out_wait(s2)
                    in_start(c + AHEAD, s2)
            else:
                if c + AHEAD < nchunk:
                    if c + AHEAD >= NBUF:
                        out_wait(s2)
                    in_start(c + AHEAD, s2)
            in_wait(s)
            compute(s)
            out_start(c, s)

        pltpu.sync_copy(idx_hbm.at[pl.ds(t_base, spw)], idx_v)
        for s in range(AHEAD):
            in_start(s, s)

        @pl.loop(0, n_outer)
        def outer(k):
            for i in range(NBUF):
                step(k * NBUF + i, i, True)

        for c in range(n_main, nchunk):
            step(c, c % NBUF, False)

        for c in range(max(0, nchunk - NBUF), nchunk):
            out_wait(c % NBUF)

    return run(x, table, idx_sc)


def _tc_body(pad_ref, x_ref, t_ref, o_ref):
    del pad_ref
    o_ref[...] = x_ref[...] + t_ref[...][:, None, :] * SCALE


def kernel(x, table, padding):
    T, B, D = x.shape
    n_rows = table.shape[0]
    t_lo = T - S_SC
    pad32 = jnp.asarray(padding, jnp.int32)
    idx_sc = jnp.clip(
        jnp.arange(t_lo, T, dtype=jnp.int32) + pad32, 0, n_rows - 1)

    sc_out = _sc_rows(x, table, idx_sc, S_SC, B, D, t_lo)

    tb = TBLK
    pad = pad32.reshape((1,))

    def x_map(i, pad_ref):
        del pad_ref
        return (i, 0, 0)

    def t_map(i, pad_ref):
        blk = jnp.minimum(i + pad_ref[0] // tb, n_rows // tb - 1)
        return (blk, 0)

    tc_out = pl.pallas_call(
        _tc_body,
        grid_spec=pltpu.PrefetchScalarGridSpec(
            num_scalar_prefetch=1,
            grid=(t_lo // tb,),
            in_specs=[
                pl.BlockSpec((tb, B, D), x_map),
                pl.BlockSpec((tb, D), t_map),
            ],
            out_specs=pl.BlockSpec((tb, B, D), x_map),
        ),
        out_shape=jax.ShapeDtypeStruct(x.shape, x.dtype),
        compiler_params=pltpu.CompilerParams(
            dimension_semantics=("arbitrary",),
        ),
    )(pad, x, table)

    return lax.dynamic_update_slice(tc_out, sc_out, (t_lo, 0, 0))
